# baseline (speedup 1.0000x reference)
import functools

import jax
import jax.numpy as jnp
from jax import lax
from jax.experimental import pallas as pl
from jax.experimental.pallas import tpu as pltpu
from jax.experimental.pallas import tpu_sc as plsc

N = 10000
E = 320000
D = 128
H = 128

NP = 10240
RBLK = 1024
NCORES = 2
NSUB = 16
NTILES = NCORES * NSUB
E_TILE = E // NTILES
C = 80
NCHUNK = E_TILE // C
CH = NP // NSUB


def _dense_body(x_ref, W1_ref, b1_ref, W2_ref, b2_ref, Wk_ref, bk_ref,
                Wq_ref, bq_ref, Wv_ref, bv_ref, Ws_ref, bg_ref, Wsc_ref,
                bsc_ref, kk_ref, qv_ref, skip_ref):
    x = x_ref[...]
    dn = (((1,), (1,)), ((), ()))
    h1 = jnp.maximum(lax.dot_general(x, W1_ref[...], dn,
                                     preferred_element_type=jnp.float32)
                     + b1_ref[...], 0.0)
    h = lax.dot_general(h1, W2_ref[...], dn,
                        preferred_element_type=jnp.float32) + b2_ref[...]
    wsc = Wsc_ref[...]
    kk_ref[...] = lax.dot_general(h, Wk_ref[...], dn,
                                  preferred_element_type=jnp.float32) + bk_ref[...]
    q = lax.dot_general(h, Wq_ref[...], dn,
                        preferred_element_type=jnp.float32) + bq_ref[...]
    v = lax.dot_general(h, Wv_ref[...], dn,
                        preferred_element_type=jnp.float32) + bv_ref[...]
    qv_ref[:, :H] = q
    qv_ref[:, H:] = v * wsc
    hs = lax.dot_general(h, Ws_ref[...], dn,
                         preferred_element_type=jnp.float32)
    c0 = jnp.sum(wsc[0] * bg_ref[...]) + bsc_ref[0]
    skip_ref[...] = (jnp.sum(hs * wsc, axis=1) + c0).reshape(1, RBLK)


def _dense_stage(xp, W1, b1, W2, b2, Wk, bk, Wq, bq, Wv, bv, Ws, bg, Wsc, bsc):
    grid = NP // RBLK
    wspec = pl.BlockSpec((H, H), lambda i: (0, 0))
    bspec = pl.BlockSpec((H,), lambda i: (0,))
    return pl.pallas_call(
        _dense_body,
        grid=(grid,),
        in_specs=[
            pl.BlockSpec((RBLK, D), lambda i: (i, 0)),
            wspec, bspec, wspec, bspec, wspec, bspec, wspec, bspec,
            wspec, bspec, wspec, bspec,
            pl.BlockSpec((1, H), lambda i: (0, 0)),
            pl.BlockSpec((1,), lambda i: (0,)),
        ],
        out_specs=[
            pl.BlockSpec((RBLK, H), lambda i: (i, 0)),
            pl.BlockSpec((RBLK, 2 * H), lambda i: (i, 0)),
            pl.BlockSpec((1, RBLK), lambda i: (0, i)),
        ],
        out_shape=[
            jax.ShapeDtypeStruct((NP, H), jnp.float32),
            jax.ShapeDtypeStruct((NP, 2 * H), jnp.float32),
            jax.ShapeDtypeStruct((1, NP), jnp.float32),
        ],
    )(xp, W1, b1, W2, b2, Wk, bk, Wq, bq, Wv, bv, Ws, bg, Wsc, bsc)


def _edge_body(kk_hbm, qv_hbm, src_hbm, dst_hbm, out_hbm,
               src_v, dst_v, agg_v, krows, qvrows, shared, tmp_v, acc_v,
               ksem0, qsem0, ksem1, qsem1):
    s = lax.axis_index("s")
    c = lax.axis_index("c")
    w = c * NSUB + s
    base_e = w * E_TILE

    def _z(i, carry):
        agg_v[pl.ds(i * 16, 16)] = jnp.zeros((16,), jnp.float32)
        return carry
    lax.fori_loop(0, NP // 16, _z, 0)

    pltpu.sync_copy(src_hbm.at[pl.ds(base_e, E_TILE)], src_v)
    pltpu.sync_copy(dst_hbm.at[pl.ds(base_e, E_TILE)], dst_v)

    lanes = lax.iota(jnp.int32, 16)
    perm8 = lax.bitwise_xor(lanes, 8)
    perm4 = lax.bitwise_xor(lanes, 4)
    perm2 = lax.bitwise_xor(lanes, 2)
    perm1 = lax.bitwise_xor(lanes, 1)

    def _hsum(v):
        for p in (perm8, perm4, perm2, perm1):
            v = v + jnp.take_along_axis(v, p, axis=0,
                                        mode=lax.GatherScatterMode.PROMISE_IN_BOUNDS)
        return v

    sems = (ksem0, qsem0, ksem1, qsem1)

    def start_chunk(g, b):
        off = pl.multiple_of(g * C, 8)
        pltpu.async_copy(kk_hbm.at[dst_v.at[pl.ds(off, C)]],
                         krows.at[b], sems[2 * b])
        pltpu.async_copy(qv_hbm.at[src_v.at[pl.ds(off, C)]],
                         qvrows.at[b], sems[2 * b + 1])

    def wait_chunk(g, b):
        off = pl.multiple_of(g * C, 8)
        pltpu.make_async_copy(kk_hbm.at[dst_v.at[pl.ds(off, C)]],
                              krows.at[b], sems[2 * b]).wait()
        pltpu.make_async_copy(qv_hbm.at[src_v.at[pl.ds(off, C)]],
                              qvrows.at[b], sems[2 * b + 1]).wait()

    def compute_chunk(g, b):
        off = pl.multiple_of(g * C, 8)
        kb = krows.at[b]
        qb = qvrows.at[b]

        def group_body(t, carry):
            base = t * 16
            dstv = dst_v[pl.ds(off + base, 16)]

            def edge_body(e, macc):
                row = base + e
                acc = jnp.zeros((16,), jnp.float32)
                for fc in range(H // 16):
                    kk16 = kb[row, pl.ds(fc * 16, 16)]
                    qq16 = qb[row, pl.ds(fc * 16, 16)]
                    vv16 = qb[row, pl.ds(H + fc * 16, 16)]
                    acc = acc + vv16 / (1.0 + jnp.exp(-(kk16 + qq16)))
                m = _hsum(acc)
                return jnp.where(lanes == e, m, macc)

            macc = lax.fori_loop(0, 16, edge_body,
                                 jnp.zeros((16,), jnp.float32), unroll=8)
            plsc.addupdate_scatter(agg_v, [dstv], macc)
            return carry
        lax.fori_loop(0, C // 16, group_body, 0)

    start_chunk(0, 0)

    def pair_body(gg, carry):
        for b in (0, 1):
            g = gg * 2 + b
            wait_chunk(g, b)
            start_chunk(g + 1, 1 - b)
            compute_chunk(g, b)
        return carry
    lax.fori_loop(0, (NCHUNK - 1) // 2, pair_body, 0)

    wait_chunk(NCHUNK - 1, 0)
    compute_chunk(NCHUNK - 1, 0)

    pltpu.sync_copy(agg_v, shared.at[s])
    plsc.subcore_barrier()

    def _zr(i, carry):
        acc_v[pl.ds(i * 16, 16)] = jnp.zeros((16,), jnp.float32)
        return carry
    lax.fori_loop(0, CH // 16, _zr, 0)
    for r in range(NSUB):
        pltpu.sync_copy(shared.at[r, pl.ds(s * CH, CH)], tmp_v)
        def _ar(i, carry):
            acc_v[pl.ds(i * 16, 16)] = (acc_v[pl.ds(i * 16, 16)]
                                        + tmp_v[pl.ds(i * 16, 16)])
            return carry
        lax.fori_loop(0, CH // 16, _ar, 0)
    pltpu.sync_copy(acc_v, out_hbm.at[c, pl.ds(s * CH, CH)])


def _edge_stage(kk, qv, srcE, dstE):
    mesh = plsc.VectorSubcoreMesh(core_axis_name="c", subcore_axis_name="s")
    fn = functools.partial(
        pl.kernel,
        out_type=jax.ShapeDtypeStruct((NCORES, NP), jnp.float32),
        mesh=mesh,
        scratch_types=[
            pltpu.VMEM((E_TILE,), jnp.int32),
            pltpu.VMEM((E_TILE,), jnp.int32),
            pltpu.VMEM((NP,), jnp.float32),
            pltpu.VMEM((2, C, H), jnp.float32),
            pltpu.VMEM((2, C, 2 * H), jnp.float32),
            pltpu.VMEM_SHARED((NSUB, NP), jnp.float32),
            pltpu.VMEM((CH,), jnp.float32),
            pltpu.VMEM((CH,), jnp.float32),
            pltpu.SemaphoreType.DMA,
            pltpu.SemaphoreType.DMA,
            pltpu.SemaphoreType.DMA,
            pltpu.SemaphoreType.DMA,
        ],
        compiler_params=pltpu.CompilerParams(needs_layout_passes=False),
    )(_edge_body)
    return fn(kk, qv, srcE, dstE)


def _final_body(p_ref, skip_ref, out_ref):
    out_ref[...] = p_ref[0:1, :] + p_ref[1:2, :] + skip_ref[...]


def _final_stage(partials, skip):
    return pl.pallas_call(
        _final_body,
        out_shape=jax.ShapeDtypeStruct((1, NP), jnp.float32),
    )(partials, skip)


def kernel(x, edge_index, W1, b1, W2, b2, Wk, bk, Wq, bq, Wv, bv, Ws, bg, Wsc, bsc):
    xp = jnp.pad(x, ((0, NP - N), (0, 0)))
    kk, qv, skip = _dense_stage(xp, W1, b1, W2, b2, Wk, bk, Wq, bq, Wv, bv,
                                Ws, bg, Wsc, bsc)
    srcE = edge_index[0]
    dstE = edge_index[1]
    partials = _edge_stage(kk, qv, srcE, dstE)
    score = _final_stage(partials, skip)
    return score[0, :N, None]


# speedup vs baseline: 2.0315x; 2.0315x over previous
import functools

import jax
import jax.numpy as jnp
from jax import lax
from jax.experimental import pallas as pl
from jax.experimental.pallas import tpu as pltpu
from jax.experimental.pallas import tpu_sc as plsc

N = 10000
E = 320000
D = 128
H = 128

NP = 10240
RBLK = 1024
NCORES = 2
NSUB = 16
NTILES = NCORES * NSUB
E_TILE = E // NTILES
C = 80
NCHUNK = E_TILE // C
CH = NP // NSUB


def _dense_body(x_ref, W1_ref, b1_ref, W2_ref, b2_ref, Wk_ref, bk_ref,
                Wq_ref, bq_ref, Wv_ref, bv_ref, Ws_ref, bg_ref, Wsc_ref,
                bsc_ref, kk_ref, qq_ref, vw_ref, skip_ref):
    x = x_ref[...]
    dn = (((1,), (1,)), ((), ()))
    h1 = jnp.maximum(lax.dot_general(x, W1_ref[...], dn,
                                     preferred_element_type=jnp.float32)
                     + b1_ref[...], 0.0)
    h = lax.dot_general(h1, W2_ref[...], dn,
                        preferred_element_type=jnp.float32) + b2_ref[...]
    wsc = Wsc_ref[...]
    kk_ref[...] = lax.dot_general(h, Wk_ref[...], dn,
                                  preferred_element_type=jnp.float32) + bk_ref[...]
    qq_ref[...] = lax.dot_general(h, Wq_ref[...], dn,
                                  preferred_element_type=jnp.float32) + bq_ref[...]
    v = lax.dot_general(h, Wv_ref[...], dn,
                        preferred_element_type=jnp.float32) + bv_ref[...]
    vw_ref[...] = v * wsc
    hs = lax.dot_general(h, Ws_ref[...], dn,
                         preferred_element_type=jnp.float32)
    c0 = jnp.sum(wsc[0] * bg_ref[...]) + bsc_ref[0]
    skip_ref[...] = (jnp.sum(hs * wsc, axis=1) + c0).reshape(1, RBLK)


def _dense_stage(xp, W1, b1, W2, b2, Wk, bk, Wq, bq, Wv, bv, Ws, bg, Wsc, bsc):
    grid = NP // RBLK
    wspec = pl.BlockSpec((H, H), lambda i: (0, 0))
    bspec = pl.BlockSpec((H,), lambda i: (0,))
    rspec = pl.BlockSpec((RBLK, H), lambda i: (i, 0))
    return pl.pallas_call(
        _dense_body,
        grid=(grid,),
        in_specs=[
            pl.BlockSpec((RBLK, D), lambda i: (i, 0)),
            wspec, bspec, wspec, bspec, wspec, bspec, wspec, bspec,
            wspec, bspec, wspec, bspec,
            pl.BlockSpec((1, H), lambda i: (0, 0)),
            pl.BlockSpec((1,), lambda i: (0,)),
        ],
        out_specs=[
            rspec, rspec, rspec,
            pl.BlockSpec((1, RBLK), lambda i: (0, i)),
        ],
        out_shape=[
            jax.ShapeDtypeStruct((NP, H), jnp.float32),
            jax.ShapeDtypeStruct((NP, H), jnp.float32),
            jax.ShapeDtypeStruct((NP, H), jnp.float32),
            jax.ShapeDtypeStruct((1, NP), jnp.float32),
        ],
    )(xp, W1, b1, W2, b2, Wk, bk, Wq, bq, Wv, bv, Ws, bg, Wsc, bsc)


def _edge_body(kk_hbm, qq_hbm, vw_hbm, src_hbm, dst_hbm, out_hbm,
               src_v, dst_v, agg_v, kqbuf, vbuf, shared, tmp_v, acc_v,
               sa0, sa1, sa2, sk0, sk1, sk2):
    s = lax.axis_index("s")
    c = lax.axis_index("c")
    w = c * NSUB + s
    base_e = w * E_TILE
    sa = (sa0, sa1, sa2)
    sk = (sk0, sk1, sk2)

    def _z(i, carry):
        agg_v[pl.ds(i * 16, 16)] = jnp.zeros((16,), jnp.float32)
        return carry
    lax.fori_loop(0, NP // 16, _z, 0)

    pltpu.sync_copy(src_hbm.at[pl.ds(base_e, E_TILE)], src_v)
    pltpu.sync_copy(dst_hbm.at[pl.ds(base_e, E_TILE)], dst_v)

    lanes = lax.iota(jnp.int32, 16)
    perm8 = lax.bitwise_xor(lanes, 8)
    perm4 = lax.bitwise_xor(lanes, 4)
    perm2 = lax.bitwise_xor(lanes, 2)
    perm1 = lax.bitwise_xor(lanes, 1)

    def _hsum(v):
        for p in (perm8, perm4, perm2, perm1):
            v = v + jnp.take_along_axis(
                v, p, axis=0, mode=lax.GatherScatterMode.PROMISE_IN_BOUNDS)
        return v

    def issue_qv(g, p):
        off = pl.multiple_of(g * C, 8)
        idx = src_v.at[pl.ds(off, C)]
        pltpu.async_copy(qq_hbm.at[idx], kqbuf.at[p], sa[p])
        pltpu.async_copy(vw_hbm.at[idx], vbuf.at[p], sa[p])

    def issue_kadd(g, p):
        off = pl.multiple_of(g * C, 8)
        idx = src_v.at[pl.ds(off, C)]
        pltpu.make_async_copy(qq_hbm.at[idx], kqbuf.at[p], sa[p]).wait()
        pltpu.make_async_copy(vw_hbm.at[idx], vbuf.at[p], sa[p]).wait()
        pltpu.async_copy(kk_hbm.at[dst_v.at[pl.ds(off, C)]],
                         kqbuf.at[p], sk[p], add=True)

    def wait_kadd(g, p):
        off = pl.multiple_of(g * C, 8)
        pltpu.make_async_copy(kk_hbm.at[dst_v.at[pl.ds(off, C)]],
                              kqbuf.at[p], sk[p]).wait()

    def compute_chunk(g, p):
        off = pl.multiple_of(g * C, 8)
        kqb = kqbuf.at[p]
        vb = vbuf.at[p]
        for t in range(C // 16):
            dstv = dst_v[pl.ds(off + t * 16, 16)]

            def edge_body(e, macc):
                row = t * 16 + e
                acc = jnp.zeros((16,), jnp.float32)
                for fc in range(H // 16):
                    t16 = kqb[row, pl.ds(fc * 16, 16)]
                    vv16 = vb[row, pl.ds(fc * 16, 16)]
                    acc = acc + vv16 / (1.0 + jnp.exp(-t16))
                m = _hsum(acc)
                return jnp.where(lanes == e, m, macc)

            macc = lax.fori_loop(0, 16, edge_body,
                                 jnp.zeros((16,), jnp.float32), unroll=4)
            plsc.addupdate_scatter(agg_v, [dstv], macc)

    issue_qv(0, 0)
    issue_qv(1, 1)
    issue_kadd(0, 0)

    def triple_body(mm, carry):
        g0 = mm * 3
        for j in (0, 1, 2):
            g = g0 + j
            issue_qv(g + 2, (j + 2) % 3)
            issue_kadd(g + 1, (j + 1) % 3)
            wait_kadd(g, j)
            compute_chunk(g, j)
        return carry
    lax.fori_loop(0, (NCHUNK - 2) // 3, triple_body, 0)

    issue_kadd(NCHUNK - 1, (NCHUNK - 1) % 3)
    wait_kadd(NCHUNK - 2, (NCHUNK - 2) % 3)
    compute_chunk(NCHUNK - 2, (NCHUNK - 2) % 3)
    wait_kadd(NCHUNK - 1, (NCHUNK - 1) % 3)
    compute_chunk(NCHUNK - 1, (NCHUNK - 1) % 3)

    pltpu.sync_copy(agg_v, shared.at[s])
    plsc.subcore_barrier()

    def _zr(i, carry):
        acc_v[pl.ds(i * 16, 16)] = jnp.zeros((16,), jnp.float32)
        return carry
    lax.fori_loop(0, CH // 16, _zr, 0)
    for r in range(NSUB):
        pltpu.sync_copy(shared.at[r, pl.ds(s * CH, CH)], tmp_v)
        def _ar(i, carry):
            acc_v[pl.ds(i * 16, 16)] = (acc_v[pl.ds(i * 16, 16)]
                                        + tmp_v[pl.ds(i * 16, 16)])
            return carry
        lax.fori_loop(0, CH // 16, _ar, 0)
    pltpu.sync_copy(acc_v, out_hbm.at[c, pl.ds(s * CH, CH)])


def _edge_stage(kk, qq, vw, srcE, dstE):
    mesh = plsc.VectorSubcoreMesh(core_axis_name="c", subcore_axis_name="s")
    fn = functools.partial(
        pl.kernel,
        out_type=jax.ShapeDtypeStruct((NCORES, NP), jnp.float32),
        mesh=mesh,
        scratch_types=[
            pltpu.VMEM((E_TILE,), jnp.int32),
            pltpu.VMEM((E_TILE,), jnp.int32),
            pltpu.VMEM((NP,), jnp.float32),
            pltpu.VMEM((3, C, H), jnp.float32),
            pltpu.VMEM((3, C, H), jnp.float32),
            pltpu.VMEM_SHARED((NSUB, NP), jnp.float32),
            pltpu.VMEM((CH,), jnp.float32),
            pltpu.VMEM((CH,), jnp.float32),
            pltpu.SemaphoreType.DMA,
            pltpu.SemaphoreType.DMA,
            pltpu.SemaphoreType.DMA,
            pltpu.SemaphoreType.DMA,
            pltpu.SemaphoreType.DMA,
            pltpu.SemaphoreType.DMA,
        ],
        compiler_params=pltpu.CompilerParams(needs_layout_passes=False),
    )(_edge_body)
    return fn(kk, qq, vw, srcE, dstE)


def _final_body(p_ref, skip_ref, out_ref):
    out_ref[...] = p_ref[0:1, :] + p_ref[1:2, :] + skip_ref[...]


def _final_stage(partials, skip):
    return pl.pallas_call(
        _final_body,
        out_shape=jax.ShapeDtypeStruct((1, NP), jnp.float32),
    )(partials, skip)


def kernel(x, edge_index, W1, b1, W2, b2, Wk, bk, Wq, bq, Wv, bv, Ws, bg, Wsc, bsc):
    xp = jnp.pad(x, ((0, NP - N), (0, 0)))
    kk, qq, vw, skip = _dense_stage(xp, W1, b1, W2, b2, Wk, bk, Wq, bq, Wv, bv,
                                    Ws, bg, Wsc, bsc)
    srcE = edge_index[0]
    dstE = edge_index[1]
    partials = _edge_stage(kk, qq, vw, srcE, dstE)
    score = _final_stage(partials, skip)
    return score[0, :N, None]
